# baseline (speedup 1.0000x reference)
import functools

import jax
import jax.numpy as jnp
from jax import lax
from jax.experimental import pallas as pl
from jax.experimental.pallas import tpu as pltpu
from jax.experimental.pallas import tpu_sc as plsc

_SC_PARAMS = pltpu.CompilerParams(needs_layout_passes=False)

NC = 2
NS = 16
NW = NC * NS
LANES = 16

N_PAD = 10240
D = 128
B = 1024


def _sc_hist(dst):
    e_w = dst.shape[0] // NW
    mesh = plsc.VectorSubcoreMesh(core_axis_name="c", subcore_axis_name="s")

    @functools.partial(
        pl.kernel,
        out_type=jax.ShapeDtypeStruct((NW, N_PAD), jnp.float32),
        mesh=mesh,
        compiler_params=_SC_PARAMS,
        scratch_types=[
            pltpu.VMEM((e_w,), jnp.int32),
            pltpu.VMEM((N_PAD,), jnp.float32),
        ],
    )
    def k(dst_hbm, out_hbm, dst_v, hist):
        cid = lax.axis_index("c")
        sid = lax.axis_index("s")
        wid = sid * NC + cid

        def zero_body(i, carry):
            hist[pl.ds(i * LANES, LANES)] = jnp.zeros((LANES,), jnp.float32)
            return carry

        lax.fori_loop(0, N_PAD // LANES, zero_body, 0)
        pltpu.sync_copy(dst_hbm.at[pl.ds(wid * e_w, e_w)], dst_v)
        ones = jnp.ones((LANES,), jnp.float32)

        def body(i, carry):
            idx = dst_v[pl.ds(i * LANES, LANES)]
            plsc.addupdate_scatter(hist, [idx], ones)
            return carry

        lax.fori_loop(0, e_w // LANES, body, 0)
        pltpu.sync_copy(hist, out_hbm.at[wid])

    return k(dst)


K = 64

E_W0 = 7424
E_W1 = 12608
E_W_MAX = max(E_W0, E_W1)
E_CAP = NS * (E_W0 + E_W1)
E_ARR = E_CAP + (E_W_MAX - min(E_W0, E_W1))


def _sc_agg(hp, src3, dst3):
    rows_t = N_PAD // NS
    mesh = plsc.VectorSubcoreMesh(core_axis_name="c", subcore_axis_name="s")

    @functools.partial(
        pl.kernel,
        out_type=jax.ShapeDtypeStruct((NC, N_PAD, D), jnp.float32),
        mesh=mesh,
        compiler_params=_SC_PARAMS,
        scratch_types=[
            pltpu.VMEM_SHARED((N_PAD, D), jnp.float32),
            pltpu.VMEM((E_W_MAX,), jnp.int32),
            pltpu.VMEM((E_W_MAX // 2,), jnp.int32),
            pltpu.VMEM((3, K, D), jnp.float32),
            pltpu.VMEM((3, 1, K), jnp.int32),
            pltpu.SemaphoreType.DMA((3,)),
            pltpu.SemaphoreType.DMA((3,)),
        ],
    )
    def k(hp_hbm, src_hbm, dst_hbm, out_hbm, acc, src_v, dst_v, rows, dbuf,
          gsem, ssem):
        cid = lax.axis_index("c")
        sid = lax.axis_index("s")
        r0 = sid * rows_t
        base = pl.multiple_of(
            jnp.where(cid == 0, sid * E_W0, NS * E_W0 + sid * E_W1), 16)
        n_chunks = jnp.where(cid == 0, E_W0 // K, E_W1 // K)
        pltpu.sync_copy(src_hbm.at[pl.ds(base, E_W_MAX)], src_v)
        pltpu.sync_copy(
            dst_hbm.at[pl.ds(pl.multiple_of(base // 2, 8), E_W_MAX // 2)],
            dst_v)
        pltpu.sync_copy(hp_hbm.at[pl.ds(r0, rows_t)], acc.at[pl.ds(r0, rows_t)])
        plsc.subcore_barrier()

        def fire_g(j, par):
            pltpu.async_copy(hp_hbm.at[src_v.at[pl.ds(j * K, K)]],
                             rows.at[par], gsem.at[par])

        def wait_g(j, par):
            pltpu.make_async_copy(hp_hbm.at[src_v.at[pl.ds(j * K, K)]],
                                  rows.at[par], gsem.at[par]).wait()

        def fire_s(par):
            pltpu.async_copy(rows.at[par], acc.at[dbuf.at[par, 0]],
                             ssem.at[par], add=True)

        def wait_s(par):
            pltpu.make_async_copy(rows.at[par], acc.at[dbuf.at[par, 0]],
                                  ssem.at[par]).wait()

        fire_g(0, 0)

        def body(j, carry):
            par = lax.rem(j, 3)

            @pl.when(j < n_chunks - 1)
            def _():
                fire_g(j + 1, lax.rem(j + 1, 3))

            wait_g(j, par)

            @pl.when(j > 0)
            def _():
                wait_s(lax.rem(j + 2, 3))

            for i in range(K // (2 * LANES)):
                ab32 = dst_v[pl.ds(j * (K // 2) + i * LANES, LANES)]
                ab = plsc.bitcast(ab32, jnp.int16)
                a, b = plsc.unpack(ab, format=plsc.PackFormat.INTERLEAVED,
                                   preferred_element_type=jnp.int32)
                dbuf[par, 0, pl.ds(i * 2 * LANES, LANES)] = a
                dbuf[par, 0, pl.ds(i * 2 * LANES + LANES, LANES)] = b
            fire_s(par)
            return carry

        lax.fori_loop(0, n_chunks, body, 0)
        wait_s(lax.rem(n_chunks - 1, 3))

        plsc.subcore_barrier()
        pltpu.sync_copy(acc.at[pl.ds(r0, rows_t)],
                        out_hbm.at[cid, pl.ds(r0, rows_t)])

    return k(hp, src3, dst3)


def _tc1(x_pad, W1, hist):
    def body(hist_ref, x_ref, w_ref, h1p_ref, dinv_ref):
        deg = jnp.sum(hist_ref[...], axis=0, keepdims=True) + 1.0
        dinv_col = lax.rsqrt(deg).reshape(B, 1)
        h = jnp.dot(x_ref[...], w_ref[...], preferred_element_type=jnp.float32)
        h1p_ref[...] = dinv_col * h
        dinv_ref[...] = dinv_col

    return pl.pallas_call(
        body,
        grid=(N_PAD // B,),
        in_specs=[
            pl.BlockSpec((NW, B), lambda i: (0, i)),
            pl.BlockSpec((B, D), lambda i: (i, 0)),
            pl.BlockSpec((D, D), lambda i: (0, 0)),
        ],
        out_specs=[
            pl.BlockSpec((B, D), lambda i: (i, 0)),
            pl.BlockSpec((B, 1), lambda i: (i, 0)),
        ],
        out_shape=[
            jax.ShapeDtypeStruct((N_PAD, D), jnp.float32),
            jax.ShapeDtypeStruct((N_PAD, 1), jnp.float32),
        ],
    )(hist, x_pad, W1)


def _tc2(p0, p1, h1p, dinv, b1, W2):
    def body(p0_ref, p1_ref, h1p_ref, dinv_ref, b_ref, w_ref, h2p_ref):
        dinv_col = dinv_ref[...]
        z = dinv_col * (p0_ref[...] + p1_ref[...] - h1p_ref[...]) + b_ref[...]
        z = jnp.maximum(z, 0.0)
        h = jnp.dot(z, w_ref[...], preferred_element_type=jnp.float32)
        h2p_ref[...] = dinv_col * h

    return pl.pallas_call(
        body,
        grid=(N_PAD // B,),
        in_specs=[
            pl.BlockSpec((B, D), lambda i: (i, 0)),
            pl.BlockSpec((B, D), lambda i: (i, 0)),
            pl.BlockSpec((B, D), lambda i: (i, 0)),
            pl.BlockSpec((B, 1), lambda i: (i, 0)),
            pl.BlockSpec((1, D), lambda i: (0, 0)),
            pl.BlockSpec((D, D), lambda i: (0, 0)),
        ],
        out_specs=pl.BlockSpec((B, D), lambda i: (i, 0)),
        out_shape=jax.ShapeDtypeStruct((N_PAD, D), jnp.float32),
    )(p0, p1, h1p, dinv, b1, W2)


def _tc3(p0, p1, h2p, dinv, b2):
    def body(p0_ref, p1_ref, h2p_ref, dinv_ref, b_ref, out_ref):
        out_ref[...] = (dinv_ref[...]
                        * (p0_ref[...] + p1_ref[...] - h2p_ref[...])
                        + b_ref[...])

    return pl.pallas_call(
        body,
        grid=(N_PAD // B,),
        in_specs=[
            pl.BlockSpec((B, D), lambda i: (i, 0)),
            pl.BlockSpec((B, D), lambda i: (i, 0)),
            pl.BlockSpec((B, D), lambda i: (i, 0)),
            pl.BlockSpec((B, 1), lambda i: (i, 0)),
            pl.BlockSpec((1, D), lambda i: (0, 0)),
        ],
        out_specs=pl.BlockSpec((B, D), lambda i: (i, 0)),
        out_shape=jax.ShapeDtypeStruct((N_PAD, D), jnp.float32),
    )(p0, p1, h2p, dinv, b2)


def kernel(x, edge_index, W1, b1, W2, b2):
    n = x.shape[0]
    ei = edge_index.astype(jnp.int32)
    src, dst = ei[0], ei[1]
    pad = jnp.full((E_ARR - src.shape[0],), N_PAD - 1, jnp.int32)
    src3 = jnp.concatenate([src, pad])
    d = jnp.concatenate([dst, pad]).reshape(-1, 2, LANES)
    dst3 = (d[:, 0] | (d[:, 1] << 16)).reshape(-1)
    x_pad = jnp.pad(x.astype(jnp.float32), ((0, N_PAD - n), (0, 0)))

    hist = _sc_hist(dst)
    h1p, dinv = _tc1(x_pad, W1.astype(jnp.float32), hist)
    p = _sc_agg(h1p, src3, dst3)
    h2p = _tc2(p[0], p[1], h1p, dinv, b1.reshape(1, D), W2.astype(jnp.float32))
    p2 = _sc_agg(h2p, src3, dst3)
    out = _tc3(p2[0], p2[1], h2p, dinv, b2.reshape(1, D))
    return out[:n]


# speedup vs baseline: 1.0006x; 1.0006x over previous
import functools

import jax
import jax.numpy as jnp
from jax import lax
from jax.experimental import pallas as pl
from jax.experimental.pallas import tpu as pltpu
from jax.experimental.pallas import tpu_sc as plsc

_SC_PARAMS = pltpu.CompilerParams(needs_layout_passes=False)

NC = 2
NS = 16
NW = NC * NS
LANES = 16

N_PAD = 10240
D = 128
B = 1024


def _sc_hist(dst):
    e_w = dst.shape[0] // NW
    mesh = plsc.VectorSubcoreMesh(core_axis_name="c", subcore_axis_name="s")

    @functools.partial(
        pl.kernel,
        out_type=jax.ShapeDtypeStruct((NW, N_PAD), jnp.float32),
        mesh=mesh,
        compiler_params=_SC_PARAMS,
        scratch_types=[
            pltpu.VMEM((e_w,), jnp.int32),
            pltpu.VMEM((N_PAD,), jnp.float32),
        ],
    )
    def k(dst_hbm, out_hbm, dst_v, hist):
        cid = lax.axis_index("c")
        sid = lax.axis_index("s")
        wid = sid * NC + cid

        def zero_body(i, carry):
            hist[pl.ds(i * LANES, LANES)] = jnp.zeros((LANES,), jnp.float32)
            return carry

        lax.fori_loop(0, N_PAD // LANES, zero_body, 0)
        pltpu.sync_copy(dst_hbm.at[pl.ds(wid * e_w, e_w)], dst_v)
        ones = jnp.ones((LANES,), jnp.float32)

        def body(i, carry):
            idx = dst_v[pl.ds(i * LANES, LANES)]
            plsc.addupdate_scatter(hist, [idx], ones)
            return carry

        lax.fori_loop(0, e_w // LANES, body, 0)
        pltpu.sync_copy(hist, out_hbm.at[wid])

    return k(dst)


K = 64

E_W0 = 7424
E_W1 = 12608
E_W_MAX = max(E_W0, E_W1)
E_CAP = NS * (E_W0 + E_W1)
E_ARR = E_CAP + (E_W_MAX - min(E_W0, E_W1))


def _sc_agg(hp, src3, dst3):
    rows_t = N_PAD // NS
    mesh = plsc.VectorSubcoreMesh(core_axis_name="c", subcore_axis_name="s")

    @functools.partial(
        pl.kernel,
        out_type=jax.ShapeDtypeStruct((NC, N_PAD, D), jnp.float32),
        mesh=mesh,
        compiler_params=_SC_PARAMS,
        scratch_types=[
            pltpu.VMEM_SHARED((N_PAD, D), jnp.float32),
            pltpu.VMEM((E_W_MAX,), jnp.int32),
            pltpu.VMEM((E_W_MAX // 2,), jnp.int32),
            pltpu.VMEM((3, K, D), jnp.float32),
            pltpu.VMEM((3, 1, K), jnp.int32),
            pltpu.SemaphoreType.DMA((3,)),
            pltpu.SemaphoreType.DMA((3,)),
        ],
    )
    def k(hp_hbm, src_hbm, dst_hbm, out_hbm, acc, src_v, dst_v, rows, dbuf,
          gsem, ssem):
        cid = lax.axis_index("c")
        sid = lax.axis_index("s")
        r0 = sid * rows_t
        base = pl.multiple_of(
            jnp.where(cid == 0, sid * E_W0, NS * E_W0 + sid * E_W1), 16)
        n_chunks = jnp.where(cid == 0, E_W0 // K, E_W1 // K)
        pltpu.sync_copy(src_hbm.at[pl.ds(base, E_W_MAX)], src_v)
        pltpu.sync_copy(
            dst_hbm.at[pl.ds(pl.multiple_of(base // 2, 8), E_W_MAX // 2)],
            dst_v)
        pltpu.sync_copy(hp_hbm.at[pl.ds(r0, rows_t)], acc.at[pl.ds(r0, rows_t)])
        plsc.subcore_barrier()

        def fire_g(j, par):
            pltpu.async_copy(hp_hbm.at[src_v.at[pl.ds(j * K, K)]],
                             rows.at[par], gsem.at[par])

        def wait_g(j, par):
            pltpu.make_async_copy(hp_hbm.at[src_v.at[pl.ds(j * K, K)]],
                                  rows.at[par], gsem.at[par]).wait()

        def fire_s(par):
            pltpu.async_copy(rows.at[par], acc.at[dbuf.at[par, 0]],
                             ssem.at[par], add=True)

        def wait_s(par):
            pltpu.make_async_copy(rows.at[par], acc.at[dbuf.at[par, 0]],
                                  ssem.at[par]).wait()

        fire_g(0, 0)

        def body(j, carry):
            par, par1, par2 = carry

            @pl.when(j < n_chunks - 1)
            def _():
                fire_g(j + 1, par1)

            wait_g(j, par)

            @pl.when(j > 0)
            def _():
                wait_s(par2)

            for i in range(K // (2 * LANES)):
                ab32 = dst_v[pl.ds(j * (K // 2) + i * LANES, LANES)]
                ab = plsc.bitcast(ab32, jnp.int16)
                a, b = plsc.unpack(ab, format=plsc.PackFormat.INTERLEAVED,
                                   preferred_element_type=jnp.int32)
                dbuf[par, 0, pl.ds(i * 2 * LANES, LANES)] = a
                dbuf[par, 0, pl.ds(i * 2 * LANES + LANES, LANES)] = b
            fire_s(par)
            return (par1, par2, par)

        last = lax.fori_loop(0, n_chunks, body,
                             (jnp.int32(0), jnp.int32(1), jnp.int32(2)))
        wait_s(last[2])

        plsc.subcore_barrier()
        pltpu.sync_copy(acc.at[pl.ds(r0, rows_t)],
                        out_hbm.at[cid, pl.ds(r0, rows_t)])

    return k(hp, src3, dst3)


def _tc1(x_pad, W1, hist):
    def body(hist_ref, x_ref, w_ref, h1p_ref, dinv_ref):
        deg = jnp.sum(hist_ref[...], axis=0, keepdims=True) + 1.0
        dinv_col = lax.rsqrt(deg).reshape(B, 1)
        h = jnp.dot(x_ref[...], w_ref[...], preferred_element_type=jnp.float32)
        h1p_ref[...] = dinv_col * h
        dinv_ref[...] = dinv_col

    return pl.pallas_call(
        body,
        grid=(N_PAD // B,),
        in_specs=[
            pl.BlockSpec((NW, B), lambda i: (0, i)),
            pl.BlockSpec((B, D), lambda i: (i, 0)),
            pl.BlockSpec((D, D), lambda i: (0, 0)),
        ],
        out_specs=[
            pl.BlockSpec((B, D), lambda i: (i, 0)),
            pl.BlockSpec((B, 1), lambda i: (i, 0)),
        ],
        out_shape=[
            jax.ShapeDtypeStruct((N_PAD, D), jnp.float32),
            jax.ShapeDtypeStruct((N_PAD, 1), jnp.float32),
        ],
    )(hist, x_pad, W1)


def _tc2(p0, p1, h1p, dinv, b1, W2):
    def body(p0_ref, p1_ref, h1p_ref, dinv_ref, b_ref, w_ref, h2p_ref):
        dinv_col = dinv_ref[...]
        z = dinv_col * (p0_ref[...] + p1_ref[...] - h1p_ref[...]) + b_ref[...]
        z = jnp.maximum(z, 0.0)
        h = jnp.dot(z, w_ref[...], preferred_element_type=jnp.float32)
        h2p_ref[...] = dinv_col * h

    return pl.pallas_call(
        body,
        grid=(N_PAD // B,),
        in_specs=[
            pl.BlockSpec((B, D), lambda i: (i, 0)),
            pl.BlockSpec((B, D), lambda i: (i, 0)),
            pl.BlockSpec((B, D), lambda i: (i, 0)),
            pl.BlockSpec((B, 1), lambda i: (i, 0)),
            pl.BlockSpec((1, D), lambda i: (0, 0)),
            pl.BlockSpec((D, D), lambda i: (0, 0)),
        ],
        out_specs=pl.BlockSpec((B, D), lambda i: (i, 0)),
        out_shape=jax.ShapeDtypeStruct((N_PAD, D), jnp.float32),
    )(p0, p1, h1p, dinv, b1, W2)


def _tc3(p0, p1, h2p, dinv, b2):
    def body(p0_ref, p1_ref, h2p_ref, dinv_ref, b_ref, out_ref):
        out_ref[...] = (dinv_ref[...]
                        * (p0_ref[...] + p1_ref[...] - h2p_ref[...])
                        + b_ref[...])

    return pl.pallas_call(
        body,
        grid=(N_PAD // B,),
        in_specs=[
            pl.BlockSpec((B, D), lambda i: (i, 0)),
            pl.BlockSpec((B, D), lambda i: (i, 0)),
            pl.BlockSpec((B, D), lambda i: (i, 0)),
            pl.BlockSpec((B, 1), lambda i: (i, 0)),
            pl.BlockSpec((1, D), lambda i: (0, 0)),
        ],
        out_specs=pl.BlockSpec((B, D), lambda i: (i, 0)),
        out_shape=jax.ShapeDtypeStruct((N_PAD, D), jnp.float32),
    )(p0, p1, h2p, dinv, b2)


def kernel(x, edge_index, W1, b1, W2, b2):
    n = x.shape[0]
    ei = edge_index.astype(jnp.int32)
    src, dst = ei[0], ei[1]
    pad = jnp.full((E_ARR - src.shape[0],), N_PAD - 1, jnp.int32)
    src3 = jnp.concatenate([src, pad])
    d = jnp.concatenate([dst, pad]).reshape(-1, 2, LANES)
    dst3 = (d[:, 0] | (d[:, 1] << 16)).reshape(-1)
    x_pad = jnp.pad(x.astype(jnp.float32), ((0, N_PAD - n), (0, 0)))

    hist = _sc_hist(dst)
    h1p, dinv = _tc1(x_pad, W1.astype(jnp.float32), hist)
    p = _sc_agg(h1p, src3, dst3)
    h2p = _tc2(p[0], p[1], h1p, dinv, b1.reshape(1, D), W2.astype(jnp.float32))
    p2 = _sc_agg(h2p, src3, dst3)
    out = _tc3(p2[0], p2[1], h2p, dinv, b2.reshape(1, D))
    return out[:n]


# speedup vs baseline: 1.0045x; 1.0039x over previous
import functools

import jax
import jax.numpy as jnp
from jax import lax
from jax.experimental import pallas as pl
from jax.experimental.pallas import tpu as pltpu
from jax.experimental.pallas import tpu_sc as plsc

_SC_PARAMS = pltpu.CompilerParams(needs_layout_passes=False)

NC = 2
NS = 16
NW = NC * NS
LANES = 16

N_PAD = 10240
D = 128
B = 1024


def _sc_hist(dst):
    e_w = dst.shape[0] // NW
    mesh = plsc.VectorSubcoreMesh(core_axis_name="c", subcore_axis_name="s")

    @functools.partial(
        pl.kernel,
        out_type=jax.ShapeDtypeStruct((NW, N_PAD), jnp.float32),
        mesh=mesh,
        compiler_params=_SC_PARAMS,
        scratch_types=[
            pltpu.VMEM((e_w,), jnp.int32),
            pltpu.VMEM((N_PAD,), jnp.float32),
        ],
    )
    def k(dst_hbm, out_hbm, dst_v, hist):
        cid = lax.axis_index("c")
        sid = lax.axis_index("s")
        wid = sid * NC + cid

        def zero_body(i, carry):
            hist[pl.ds(i * LANES, LANES)] = jnp.zeros((LANES,), jnp.float32)
            return carry

        lax.fori_loop(0, N_PAD // LANES, zero_body, 0)
        pltpu.sync_copy(dst_hbm.at[pl.ds(wid * e_w, e_w)], dst_v)
        ones = jnp.ones((LANES,), jnp.float32)

        def body(i, carry):
            idx = dst_v[pl.ds(i * LANES, LANES)]
            plsc.addupdate_scatter(hist, [idx], ones)
            return carry

        lax.fori_loop(0, e_w // LANES, body, 0)
        pltpu.sync_copy(hist, out_hbm.at[wid])

    return k(dst)


K = 64

E_W0 = 7424
E_W1 = 12608
E_W_MAX = max(E_W0, E_W1)
E_CAP = NS * (E_W0 + E_W1)
E_ARR = E_CAP + (E_W_MAX - min(E_W0, E_W1))


def _sc_agg(hp, src3, dst3):
    rows_t = N_PAD // NS
    mesh = plsc.VectorSubcoreMesh(core_axis_name="c", subcore_axis_name="s")

    @functools.partial(
        pl.kernel,
        out_type=jax.ShapeDtypeStruct((NC, N_PAD, D), jnp.float32),
        mesh=mesh,
        compiler_params=_SC_PARAMS,
        scratch_types=[
            pltpu.VMEM_SHARED((N_PAD, D), jnp.float32),
            pltpu.VMEM((E_W_MAX,), jnp.int32),
            pltpu.VMEM((E_W_MAX // 2,), jnp.int32),
            pltpu.VMEM((3, K, D), jnp.float32),
            pltpu.VMEM((3, 1, K), jnp.int32),
            pltpu.SemaphoreType.DMA((3,)),
            pltpu.SemaphoreType.DMA((3,)),
        ],
    )
    def k(hp_hbm, src_hbm, dst_hbm, out_hbm, acc, src_v, dst_v, rows, dbuf,
          gsem, ssem):
        cid = lax.axis_index("c")
        sid = lax.axis_index("s")
        r0 = sid * rows_t
        base = pl.multiple_of(
            jnp.where(cid == 0, sid * E_W0, NS * E_W0 + sid * E_W1), 16)
        n_chunks = jnp.where(cid == 0, E_W0 // K, E_W1 // K)
        pltpu.sync_copy(src_hbm.at[pl.ds(base, E_W_MAX)], src_v)
        pltpu.sync_copy(
            dst_hbm.at[pl.ds(pl.multiple_of(base // 2, 8), E_W_MAX // 2)],
            dst_v)
        pltpu.sync_copy(hp_hbm.at[pl.ds(r0, rows_t)], acc.at[pl.ds(r0, rows_t)])
        plsc.subcore_barrier()

        def fire_g(j, par):
            pltpu.async_copy(hp_hbm.at[src_v.at[pl.ds(j * K, K)]],
                             rows.at[par], gsem.at[par])

        def wait_g(j, par):
            pltpu.make_async_copy(hp_hbm.at[src_v.at[pl.ds(j * K, K)]],
                                  rows.at[par], gsem.at[par]).wait()

        def fire_s(par):
            pltpu.async_copy(rows.at[par], acc.at[dbuf.at[par, 0]],
                             ssem.at[par], add=True)

        def wait_s(par):
            pltpu.make_async_copy(rows.at[par], acc.at[dbuf.at[par, 0]],
                                  ssem.at[par]).wait()

        fire_g(0, 0)
        fire_g(1, 1)

        def body(j, carry):
            par, par1, par2 = carry

            wait_g(j, par)

            @pl.when(j > 0)
            def _():
                wait_s(par2)

            @pl.when(j < n_chunks - 2)
            def _():
                fire_g(j + 2, par2)

            for i in range(K // (2 * LANES)):
                ab32 = dst_v[pl.ds(j * (K // 2) + i * LANES, LANES)]
                ab = plsc.bitcast(ab32, jnp.int16)
                a, b = plsc.unpack(ab, format=plsc.PackFormat.INTERLEAVED,
                                   preferred_element_type=jnp.int32)
                dbuf[par, 0, pl.ds(i * 2 * LANES, LANES)] = a
                dbuf[par, 0, pl.ds(i * 2 * LANES + LANES, LANES)] = b
            fire_s(par)
            return (par1, par2, par)

        last = lax.fori_loop(0, n_chunks, body,
                             (jnp.int32(0), jnp.int32(1), jnp.int32(2)))
        wait_s(last[2])

        plsc.subcore_barrier()
        pltpu.sync_copy(acc.at[pl.ds(r0, rows_t)],
                        out_hbm.at[cid, pl.ds(r0, rows_t)])

    return k(hp, src3, dst3)


def _tc1(x_pad, W1, hist):
    def body(hist_ref, x_ref, w_ref, h1p_ref, dinv_ref):
        deg = jnp.sum(hist_ref[...], axis=0, keepdims=True) + 1.0
        dinv_col = lax.rsqrt(deg).reshape(B, 1)
        h = jnp.dot(x_ref[...], w_ref[...], preferred_element_type=jnp.float32)
        h1p_ref[...] = dinv_col * h
        dinv_ref[...] = dinv_col

    return pl.pallas_call(
        body,
        grid=(N_PAD // B,),
        in_specs=[
            pl.BlockSpec((NW, B), lambda i: (0, i)),
            pl.BlockSpec((B, D), lambda i: (i, 0)),
            pl.BlockSpec((D, D), lambda i: (0, 0)),
        ],
        out_specs=[
            pl.BlockSpec((B, D), lambda i: (i, 0)),
            pl.BlockSpec((B, 1), lambda i: (i, 0)),
        ],
        out_shape=[
            jax.ShapeDtypeStruct((N_PAD, D), jnp.float32),
            jax.ShapeDtypeStruct((N_PAD, 1), jnp.float32),
        ],
    )(hist, x_pad, W1)


def _tc2(p0, p1, h1p, dinv, b1, W2):
    def body(p0_ref, p1_ref, h1p_ref, dinv_ref, b_ref, w_ref, h2p_ref):
        dinv_col = dinv_ref[...]
        z = dinv_col * (p0_ref[...] + p1_ref[...] - h1p_ref[...]) + b_ref[...]
        z = jnp.maximum(z, 0.0)
        h = jnp.dot(z, w_ref[...], preferred_element_type=jnp.float32)
        h2p_ref[...] = dinv_col * h

    return pl.pallas_call(
        body,
        grid=(N_PAD // B,),
        in_specs=[
            pl.BlockSpec((B, D), lambda i: (i, 0)),
            pl.BlockSpec((B, D), lambda i: (i, 0)),
            pl.BlockSpec((B, D), lambda i: (i, 0)),
            pl.BlockSpec((B, 1), lambda i: (i, 0)),
            pl.BlockSpec((1, D), lambda i: (0, 0)),
            pl.BlockSpec((D, D), lambda i: (0, 0)),
        ],
        out_specs=pl.BlockSpec((B, D), lambda i: (i, 0)),
        out_shape=jax.ShapeDtypeStruct((N_PAD, D), jnp.float32),
    )(p0, p1, h1p, dinv, b1, W2)


def _tc3(p0, p1, h2p, dinv, b2):
    def body(p0_ref, p1_ref, h2p_ref, dinv_ref, b_ref, out_ref):
        out_ref[...] = (dinv_ref[...]
                        * (p0_ref[...] + p1_ref[...] - h2p_ref[...])
                        + b_ref[...])

    return pl.pallas_call(
        body,
        grid=(N_PAD // B,),
        in_specs=[
            pl.BlockSpec((B, D), lambda i: (i, 0)),
            pl.BlockSpec((B, D), lambda i: (i, 0)),
            pl.BlockSpec((B, D), lambda i: (i, 0)),
            pl.BlockSpec((B, 1), lambda i: (i, 0)),
            pl.BlockSpec((1, D), lambda i: (0, 0)),
        ],
        out_specs=pl.BlockSpec((B, D), lambda i: (i, 0)),
        out_shape=jax.ShapeDtypeStruct((N_PAD, D), jnp.float32),
    )(p0, p1, h2p, dinv, b2)


def kernel(x, edge_index, W1, b1, W2, b2):
    n = x.shape[0]
    ei = edge_index.astype(jnp.int32)
    src, dst = ei[0], ei[1]
    pad = jnp.full((E_ARR - src.shape[0],), N_PAD - 1, jnp.int32)
    src3 = jnp.concatenate([src, pad])
    d = jnp.concatenate([dst, pad]).reshape(-1, 2, LANES)
    dst3 = (d[:, 0] | (d[:, 1] << 16)).reshape(-1)
    x_pad = jnp.pad(x.astype(jnp.float32), ((0, N_PAD - n), (0, 0)))

    hist = _sc_hist(dst)
    h1p, dinv = _tc1(x_pad, W1.astype(jnp.float32), hist)
    p = _sc_agg(h1p, src3, dst3)
    h2p = _tc2(p[0], p[1], h1p, dinv, b1.reshape(1, D), W2.astype(jnp.float32))
    p2 = _sc_agg(h2p, src3, dst3)
    out = _tc3(p2[0], p2[1], h2p, dinv, b2.reshape(1, D))
    return out[:n]


# speedup vs baseline: 1.0533x; 1.0485x over previous
import functools

import jax
import jax.numpy as jnp
from jax import lax
from jax.experimental import pallas as pl
from jax.experimental.pallas import tpu as pltpu
from jax.experimental.pallas import tpu_sc as plsc

_SC_PARAMS = pltpu.CompilerParams(needs_layout_passes=False)

NC = 2
NS = 16
NW = NC * NS
LANES = 16

N_PAD = 10240
D = 128
B = 1024


def _sc_hist(dst):
    e_w = dst.shape[0] // NW
    mesh = plsc.VectorSubcoreMesh(core_axis_name="c", subcore_axis_name="s")

    @functools.partial(
        pl.kernel,
        out_type=jax.ShapeDtypeStruct((NW, N_PAD), jnp.float32),
        mesh=mesh,
        compiler_params=_SC_PARAMS,
        scratch_types=[
            pltpu.VMEM((e_w,), jnp.int32),
            pltpu.VMEM((N_PAD,), jnp.float32),
        ],
    )
    def k(dst_hbm, out_hbm, dst_v, hist):
        cid = lax.axis_index("c")
        sid = lax.axis_index("s")
        wid = sid * NC + cid

        def zero_body(i, carry):
            hist[pl.ds(i * LANES, LANES)] = jnp.zeros((LANES,), jnp.float32)
            return carry

        lax.fori_loop(0, N_PAD // LANES, zero_body, 0)
        pltpu.sync_copy(dst_hbm.at[pl.ds(wid * e_w, e_w)], dst_v)
        ones = jnp.ones((LANES,), jnp.float32)

        def body(i, carry):
            idx = dst_v[pl.ds(i * LANES, LANES)]
            plsc.addupdate_scatter(hist, [idx], ones)
            return carry

        lax.fori_loop(0, e_w // LANES, body, 0)
        pltpu.sync_copy(hist, out_hbm.at[wid])

    return k(dst)


K = 64

E_W0 = 12608
E_W1 = 7424
E_W_MAX = max(E_W0, E_W1)
E_CAP = NS * (E_W0 + E_W1)
E_ARR = E_CAP + (E_W_MAX - min(E_W0, E_W1))


def _sc_agg(hp, src3, dst3):
    rows_t = N_PAD // NS
    mesh = plsc.VectorSubcoreMesh(core_axis_name="c", subcore_axis_name="s")

    @functools.partial(
        pl.kernel,
        out_type=jax.ShapeDtypeStruct((NC, N_PAD, D), jnp.float32),
        mesh=mesh,
        compiler_params=_SC_PARAMS,
        scratch_types=[
            pltpu.VMEM_SHARED((N_PAD, D), jnp.float32),
            pltpu.VMEM((E_W_MAX,), jnp.int32),
            pltpu.VMEM((E_W_MAX // 2,), jnp.int32),
            pltpu.VMEM((3, K, D), jnp.float32),
            pltpu.VMEM((3, 1, K), jnp.int32),
            pltpu.SemaphoreType.DMA((3,)),
            pltpu.SemaphoreType.DMA((3,)),
        ],
    )
    def k(hp_hbm, src_hbm, dst_hbm, out_hbm, acc, src_v, dst_v, rows, dbuf,
          gsem, ssem):
        cid = lax.axis_index("c")
        sid = lax.axis_index("s")
        r0 = sid * rows_t
        base = pl.multiple_of(
            jnp.where(cid == 0, sid * E_W0, NS * E_W0 + sid * E_W1), 16)
        n_chunks = jnp.where(cid == 0, E_W0 // K, E_W1 // K)
        pltpu.sync_copy(src_hbm.at[pl.ds(base, E_W_MAX)], src_v)
        pltpu.sync_copy(
            dst_hbm.at[pl.ds(pl.multiple_of(base // 2, 8), E_W_MAX // 2)],
            dst_v)
        pltpu.sync_copy(hp_hbm.at[pl.ds(r0, rows_t)], acc.at[pl.ds(r0, rows_t)])
        plsc.subcore_barrier()

        def fire_g(j, par):
            pltpu.async_copy(hp_hbm.at[src_v.at[pl.ds(j * K, K)]],
                             rows.at[par], gsem.at[par])

        def wait_g(j, par):
            pltpu.make_async_copy(hp_hbm.at[src_v.at[pl.ds(j * K, K)]],
                                  rows.at[par], gsem.at[par]).wait()

        def fire_s(par):
            pltpu.async_copy(rows.at[par], acc.at[dbuf.at[par, 0]],
                             ssem.at[par], add=True)

        def wait_s(par):
            pltpu.make_async_copy(rows.at[par], acc.at[dbuf.at[par, 0]],
                                  ssem.at[par]).wait()

        fire_g(0, 0)
        fire_g(1, 1)

        def body(j, carry):
            par, par1, par2 = carry

            wait_g(j, par)

            @pl.when(j > 0)
            def _():
                wait_s(par2)

            @pl.when(j < n_chunks - 2)
            def _():
                fire_g(j + 2, par2)

            for i in range(K // (2 * LANES)):
                ab32 = dst_v[pl.ds(j * (K // 2) + i * LANES, LANES)]
                ab = plsc.bitcast(ab32, jnp.int16)
                a, b = plsc.unpack(ab, format=plsc.PackFormat.INTERLEAVED,
                                   preferred_element_type=jnp.int32)
                dbuf[par, 0, pl.ds(i * 2 * LANES, LANES)] = a
                dbuf[par, 0, pl.ds(i * 2 * LANES + LANES, LANES)] = b
            fire_s(par)
            return (par1, par2, par)

        last = lax.fori_loop(0, n_chunks, body,
                             (jnp.int32(0), jnp.int32(1), jnp.int32(2)))
        wait_s(last[2])

        plsc.subcore_barrier()
        pltpu.sync_copy(acc.at[pl.ds(r0, rows_t)],
                        out_hbm.at[cid, pl.ds(r0, rows_t)])

    return k(hp, src3, dst3)


def _tc1(x_pad, W1, hist):
    def body(hist_ref, x_ref, w_ref, h1p_ref, dinv_ref):
        deg = jnp.sum(hist_ref[...], axis=0, keepdims=True) + 1.0
        dinv_col = lax.rsqrt(deg).reshape(B, 1)
        h = jnp.dot(x_ref[...], w_ref[...], preferred_element_type=jnp.float32)
        h1p_ref[...] = dinv_col * h
        dinv_ref[...] = dinv_col

    return pl.pallas_call(
        body,
        grid=(N_PAD // B,),
        in_specs=[
            pl.BlockSpec((NW, B), lambda i: (0, i)),
            pl.BlockSpec((B, D), lambda i: (i, 0)),
            pl.BlockSpec((D, D), lambda i: (0, 0)),
        ],
        out_specs=[
            pl.BlockSpec((B, D), lambda i: (i, 0)),
            pl.BlockSpec((B, 1), lambda i: (i, 0)),
        ],
        out_shape=[
            jax.ShapeDtypeStruct((N_PAD, D), jnp.float32),
            jax.ShapeDtypeStruct((N_PAD, 1), jnp.float32),
        ],
    )(hist, x_pad, W1)


def _tc2(p0, p1, h1p, dinv, b1, W2):
    def body(p0_ref, p1_ref, h1p_ref, dinv_ref, b_ref, w_ref, h2p_ref):
        dinv_col = dinv_ref[...]
        z = dinv_col * (p0_ref[...] + p1_ref[...] - h1p_ref[...]) + b_ref[...]
        z = jnp.maximum(z, 0.0)
        h = jnp.dot(z, w_ref[...], preferred_element_type=jnp.float32)
        h2p_ref[...] = dinv_col * h

    return pl.pallas_call(
        body,
        grid=(N_PAD // B,),
        in_specs=[
            pl.BlockSpec((B, D), lambda i: (i, 0)),
            pl.BlockSpec((B, D), lambda i: (i, 0)),
            pl.BlockSpec((B, D), lambda i: (i, 0)),
            pl.BlockSpec((B, 1), lambda i: (i, 0)),
            pl.BlockSpec((1, D), lambda i: (0, 0)),
            pl.BlockSpec((D, D), lambda i: (0, 0)),
        ],
        out_specs=pl.BlockSpec((B, D), lambda i: (i, 0)),
        out_shape=jax.ShapeDtypeStruct((N_PAD, D), jnp.float32),
    )(p0, p1, h1p, dinv, b1, W2)


def _tc3(p0, p1, h2p, dinv, b2):
    def body(p0_ref, p1_ref, h2p_ref, dinv_ref, b_ref, out_ref):
        out_ref[...] = (dinv_ref[...]
                        * (p0_ref[...] + p1_ref[...] - h2p_ref[...])
                        + b_ref[...])

    return pl.pallas_call(
        body,
        grid=(N_PAD // B,),
        in_specs=[
            pl.BlockSpec((B, D), lambda i: (i, 0)),
            pl.BlockSpec((B, D), lambda i: (i, 0)),
            pl.BlockSpec((B, D), lambda i: (i, 0)),
            pl.BlockSpec((B, 1), lambda i: (i, 0)),
            pl.BlockSpec((1, D), lambda i: (0, 0)),
        ],
        out_specs=pl.BlockSpec((B, D), lambda i: (i, 0)),
        out_shape=jax.ShapeDtypeStruct((N_PAD, D), jnp.float32),
    )(p0, p1, h2p, dinv, b2)


def kernel(x, edge_index, W1, b1, W2, b2):
    n = x.shape[0]
    ei = edge_index.astype(jnp.int32)
    src, dst = ei[0], ei[1]
    pad = jnp.full((E_ARR - src.shape[0],), N_PAD - 1, jnp.int32)
    src3 = jnp.concatenate([src, pad])
    d = jnp.concatenate([dst, pad]).reshape(-1, 2, LANES)
    dst3 = (d[:, 0] | (d[:, 1] << 16)).reshape(-1)
    x_pad = jnp.pad(x.astype(jnp.float32), ((0, N_PAD - n), (0, 0)))

    hist = _sc_hist(dst)
    h1p, dinv = _tc1(x_pad, W1.astype(jnp.float32), hist)
    p = _sc_agg(h1p, src3, dst3)
    h2p = _tc2(p[0], p[1], h1p, dinv, b1.reshape(1, D), W2.astype(jnp.float32))
    p2 = _sc_agg(h2p, src3, dst3)
    out = _tc3(p2[0], p2[1], h2p, dinv, b2.reshape(1, D))
    return out[:n]
